# Initial kernel scaffold; baseline (speedup 1.0000x reference)
import jax
import jax.numpy as jnp
from jax.experimental import pallas as pl


def kernel(h, c, child_idx, U_f, b_f, U_iou, b_iou):
    raise NotImplementedError("write your pallas kernel here")


# speedup vs baseline: 1.6689x; 1.6689x over previous
import functools

import jax
import jax.numpy as jnp
from jax import lax
from jax.experimental import pallas as pl
from jax.experimental.pallas import tpu as pltpu
from jax.experimental.pallas import tpu_sc as plsc

NC = 2
NS = 16
NW = NC * NS
CH = 112


def _sc_gather_body(n_ch, idxl_hbm, idxr_hbm, h_hbm, c_hbm,
                    hl_out, hr_out, cl_out, cr_out,
                    idxl_v, idxr_v, bhl, bhr, bcl, bcr, sem):
    wid = lax.axis_index("s") * NC + lax.axis_index("c")
    base = wid * (n_ch * CH)
    for j in range(n_ch):
        off = base + j * CH
        pltpu.sync_copy(idxl_hbm.at[pl.ds(off, CH)], idxl_v)
        pltpu.sync_copy(idxr_hbm.at[pl.ds(off, CH)], idxr_v)
        g0 = pltpu.async_copy(h_hbm.at[idxl_v], bhl, sem)
        g1 = pltpu.async_copy(h_hbm.at[idxr_v], bhr, sem)
        g2 = pltpu.async_copy(c_hbm.at[idxl_v], bcl, sem)
        g3 = pltpu.async_copy(c_hbm.at[idxr_v], bcr, sem)
        g0.wait()
        g1.wait()
        g2.wait()
        g3.wait()
        pltpu.sync_copy(bhl, hl_out.at[pl.ds(off, CH)])
        pltpu.sync_copy(bhr, hr_out.at[pl.ds(off, CH)])
        pltpu.sync_copy(bcl, cl_out.at[pl.ds(off, CH)])
        pltpu.sync_copy(bcr, cr_out.at[pl.ds(off, CH)])


def _sc_gather(idxl, idxr, h, c, n_ch):
    Hs = h.shape[1]
    P_pad = idxl.shape[0]
    out = jax.ShapeDtypeStruct((P_pad, Hs), jnp.float32)
    mesh = plsc.VectorSubcoreMesh(core_axis_name="c", subcore_axis_name="s")
    row = pltpu.VMEM((CH, Hs), jnp.float32)
    fn = pl.kernel(
        functools.partial(_sc_gather_body, n_ch),
        out_type=(out, out, out, out),
        mesh=mesh,
        scratch_types=[
            pltpu.VMEM((CH,), jnp.int32),
            pltpu.VMEM((CH,), jnp.int32),
            row, row, row, row,
            pltpu.SemaphoreType.DMA,
        ],
    )
    return fn(idxl, idxr, h, c)


def _tc_body(Hs, hl_ref, hr_ref, cl_ref, cr_ref, W_ref, b_ref, ho_ref, co_ref):
    hl = hl_ref[...]
    hr = hr_ref[...]
    z = (jnp.dot(hl, W_ref[:Hs, :], preferred_element_type=jnp.float32)
         + jnp.dot(hr, W_ref[Hs:, :], preferred_element_type=jnp.float32)
         + b_ref[...])
    f = jax.nn.sigmoid(z[:, :2 * Hs])
    c_sum = f[:, :Hs] * cl_ref[...] + f[:, Hs:] * cr_ref[...]
    i = jax.nn.sigmoid(z[:, 2 * Hs:3 * Hs])
    o = jax.nn.sigmoid(z[:, 3 * Hs:4 * Hs])
    u = jnp.tanh(z[:, 4 * Hs:])
    c_out = i * u + c_sum
    co_ref[...] = c_out
    ho_ref[...] = o * jnp.tanh(c_out)


def _tc_dense(hl, hr, cl, cr, W, b, bp):
    P_pad, Hs = hl.shape
    grid = P_pad // bp
    blk = pl.BlockSpec((bp, Hs), lambda i: (i, 0))
    wblk = pl.BlockSpec(W.shape, lambda i: (0, 0))
    bblk = pl.BlockSpec(b.shape, lambda i: (0, 0))
    out = jax.ShapeDtypeStruct((P_pad, Hs), jnp.float32)
    return pl.pallas_call(
        functools.partial(_tc_body, Hs),
        grid=(grid,),
        in_specs=[blk, blk, blk, blk, wblk, bblk],
        out_specs=[blk, blk],
        out_shape=(out, out),
    )(hl, hr, cl, cr, W, b)


def kernel(h, c, child_idx, U_f, b_f, U_iou, b_iou):
    P = child_idx.shape[0]
    idx = child_idx.astype(jnp.int32)
    step = NW * CH
    P_pad = ((P + step - 1) // step) * step
    n_ch = P_pad // step
    pad = P_pad - P
    idxl = jnp.pad(idx[:, 0], (0, pad))
    idxr = jnp.pad(idx[:, 1], (0, pad))

    hl, hr, cl, cr = _sc_gather(idxl, idxr, h, c, n_ch)

    W = jnp.concatenate([U_f, U_iou], axis=1)
    b = jnp.concatenate([b_f, b_iou.reshape(-1)]).reshape(1, -1)
    bp = 1024
    while P_pad % bp != 0:
        bp //= 2
    ho, co = _tc_dense(hl, hr, cl, cr, W, b, bp)
    return ho[:P], co[:P]


# speedup vs baseline: 2.1075x; 1.2629x over previous
import functools

import jax
import jax.numpy as jnp
from jax import lax
from jax.experimental import pallas as pl
from jax.experimental.pallas import tpu as pltpu
from jax.experimental.pallas import tpu_sc as plsc

NC = 2
NS = 16
NW = NC * NS
CH = 112


def _sc_gather_body(n_ch, idxl_hbm, idxr_hbm, h_hbm, c_hbm,
                    hl_out, hr_out, cl_out, cr_out,
                    idxl0, idxr0, bhl0, bhr0, bcl0, bcr0,
                    idxl1, idxr1, bhl1, bhr1, bcl1, bcr1,
                    sem0, sem1):
    wid = lax.axis_index("s") * NC + lax.axis_index("c")
    base = wid * (n_ch * CH)
    slots = [(idxl0, idxr0, bhl0, bhr0, bcl0, bcr0, sem0),
             (idxl1, idxr1, bhl1, bhr1, bcl1, bcr1, sem1)]
    g_desc = [None, None]
    w_desc = [None, None]

    def load_idx_and_gather(j):
        idxl_v, idxr_v, bhl, bhr, bcl, bcr, sem = slots[j % 2]
        off = base + j * CH
        pltpu.sync_copy(idxl_hbm.at[pl.ds(off, CH)], idxl_v)
        pltpu.sync_copy(idxr_hbm.at[pl.ds(off, CH)], idxr_v)
        g_desc[j % 2] = [
            pltpu.async_copy(h_hbm.at[idxl_v], bhl, sem),
            pltpu.async_copy(h_hbm.at[idxr_v], bhr, sem),
            pltpu.async_copy(c_hbm.at[idxl_v], bcl, sem),
            pltpu.async_copy(c_hbm.at[idxr_v], bcr, sem),
        ]

    def issue_writeback(j):
        _, _, bhl, bhr, bcl, bcr, sem = slots[j % 2]
        off = base + j * CH
        w_desc[j % 2] = [
            pltpu.async_copy(bhl, hl_out.at[pl.ds(off, CH)], sem),
            pltpu.async_copy(bhr, hr_out.at[pl.ds(off, CH)], sem),
            pltpu.async_copy(bcl, cl_out.at[pl.ds(off, CH)], sem),
            pltpu.async_copy(bcr, cr_out.at[pl.ds(off, CH)], sem),
        ]

    load_idx_and_gather(0)
    for j in range(n_ch):
        if j + 1 < n_ch:
            if j >= 1:
                for d in w_desc[(j - 1) % 2]:
                    d.wait()
            load_idx_and_gather(j + 1)
        for d in g_desc[j % 2]:
            d.wait()
        issue_writeback(j)
    if n_ch >= 2:
        for d in w_desc[(n_ch - 2) % 2]:
            d.wait()
    for d in w_desc[(n_ch - 1) % 2]:
        d.wait()


def _sc_gather(idxl, idxr, h, c, n_ch):
    Hs = h.shape[1]
    P_pad = idxl.shape[0]
    out = jax.ShapeDtypeStruct((P_pad, Hs), jnp.float32)
    mesh = plsc.VectorSubcoreMesh(core_axis_name="c", subcore_axis_name="s")
    idxs = pltpu.VMEM((CH,), jnp.int32)
    row = pltpu.VMEM((CH, Hs), jnp.float32)
    fn = pl.kernel(
        functools.partial(_sc_gather_body, n_ch),
        out_type=(out, out, out, out),
        mesh=mesh,
        scratch_types=[
            idxs, idxs, row, row, row, row,
            idxs, idxs, row, row, row, row,
            pltpu.SemaphoreType.DMA,
            pltpu.SemaphoreType.DMA,
        ],
    )
    return fn(idxl, idxr, h, c)


def _tc_body(Hs, hl_ref, hr_ref, cl_ref, cr_ref, W_ref, b_ref, ho_ref, co_ref):
    hl = hl_ref[...]
    hr = hr_ref[...]
    z = (jnp.dot(hl, W_ref[:Hs, :], preferred_element_type=jnp.float32)
         + jnp.dot(hr, W_ref[Hs:, :], preferred_element_type=jnp.float32)
         + b_ref[...])
    f = jax.nn.sigmoid(z[:, :2 * Hs])
    c_sum = f[:, :Hs] * cl_ref[...] + f[:, Hs:] * cr_ref[...]
    i = jax.nn.sigmoid(z[:, 2 * Hs:3 * Hs])
    o = jax.nn.sigmoid(z[:, 3 * Hs:4 * Hs])
    u = jnp.tanh(z[:, 4 * Hs:])
    c_out = i * u + c_sum
    co_ref[...] = c_out
    ho_ref[...] = o * jnp.tanh(c_out)


def _tc_dense(hl, hr, cl, cr, W, b, P, bp):
    Hs = hl.shape[1]
    grid = P // bp
    blk = pl.BlockSpec((bp, Hs), lambda i: (i, 0))
    wblk = pl.BlockSpec(W.shape, lambda i: (0, 0))
    bblk = pl.BlockSpec(b.shape, lambda i: (0, 0))
    out = jax.ShapeDtypeStruct((P, Hs), jnp.float32)
    return pl.pallas_call(
        functools.partial(_tc_body, Hs),
        grid=(grid,),
        in_specs=[blk, blk, blk, blk, wblk, bblk],
        out_specs=[blk, blk],
        out_shape=(out, out),
    )(hl, hr, cl, cr, W, b)


def kernel(h, c, child_idx, U_f, b_f, U_iou, b_iou):
    P = child_idx.shape[0]
    idx = child_idx.astype(jnp.int32)
    step = NW * CH
    P_pad = ((P + step - 1) // step) * step
    n_ch = P_pad // step
    pad = P_pad - P
    idxl = jnp.pad(idx[:, 0], (0, pad))
    idxr = jnp.pad(idx[:, 1], (0, pad))

    hl, hr, cl, cr = _sc_gather(idxl, idxr, h, c, n_ch)

    W = jnp.concatenate([U_f, U_iou], axis=1)
    b = jnp.concatenate([b_f, b_iou.reshape(-1)]).reshape(1, -1)
    bp = 1000
    while P % bp != 0:
        bp //= 2
    return _tc_dense(hl, hr, cl, cr, W, b, P, bp)


# speedup vs baseline: 2.1413x; 1.0160x over previous
import functools

import jax
import jax.numpy as jnp
from jax import lax
from jax.experimental import pallas as pl
from jax.experimental.pallas import tpu as pltpu
from jax.experimental.pallas import tpu_sc as plsc

NC = 2
NS = 16
NW = NC * NS
CH = 112


def _sc_gather_body(n_ch, idxl_hbm, idxr_hbm, h_hbm, c_hbm,
                    hl_out, hr_out, cl_out, cr_out,
                    idxl_v, idxr_v,
                    bhl0, bhr0, bcl0, bcr0,
                    bhl1, bhr1, bcl1, bcr1,
                    sem0, sem1):
    wid = lax.axis_index("s") * NC + lax.axis_index("c")
    base = wid * (n_ch * CH)
    pltpu.sync_copy(idxl_hbm.at[wid], idxl_v)
    pltpu.sync_copy(idxr_hbm.at[wid], idxr_v)
    slots = [(bhl0, bhr0, bcl0, bcr0, sem0),
             (bhl1, bhr1, bcl1, bcr1, sem1)]
    g_desc = [None, None]
    w_desc = [None, None]

    def load_idx_and_gather(j):
        bhl, bhr, bcl, bcr, sem = slots[j % 2]
        g_desc[j % 2] = [
            pltpu.async_copy(h_hbm.at[idxl_v.at[j]], bhl, sem),
            pltpu.async_copy(h_hbm.at[idxr_v.at[j]], bhr, sem),
            pltpu.async_copy(c_hbm.at[idxl_v.at[j]], bcl, sem),
            pltpu.async_copy(c_hbm.at[idxr_v.at[j]], bcr, sem),
        ]

    def issue_writeback(j):
        bhl, bhr, bcl, bcr, sem = slots[j % 2]
        off = base + j * CH
        w_desc[j % 2] = [
            pltpu.async_copy(bhl, hl_out.at[pl.ds(off, CH)], sem),
            pltpu.async_copy(bhr, hr_out.at[pl.ds(off, CH)], sem),
            pltpu.async_copy(bcl, cl_out.at[pl.ds(off, CH)], sem),
            pltpu.async_copy(bcr, cr_out.at[pl.ds(off, CH)], sem),
        ]

    load_idx_and_gather(0)
    for j in range(n_ch):
        if j + 1 < n_ch:
            if j >= 1:
                for d in w_desc[(j - 1) % 2]:
                    d.wait()
            load_idx_and_gather(j + 1)
        for d in g_desc[j % 2]:
            d.wait()
        issue_writeback(j)
    if n_ch >= 2:
        for d in w_desc[(n_ch - 2) % 2]:
            d.wait()
    for d in w_desc[(n_ch - 1) % 2]:
        d.wait()


def _sc_gather(idxl, idxr, h, c, n_ch):
    Hs = h.shape[1]
    P_pad = idxl.shape[0]
    out = jax.ShapeDtypeStruct((P_pad, Hs), jnp.float32)
    mesh = plsc.VectorSubcoreMesh(core_axis_name="c", subcore_axis_name="s")
    idxs = pltpu.VMEM((n_ch, CH), jnp.int32)
    row = pltpu.VMEM((CH, Hs), jnp.float32)
    fn = pl.kernel(
        functools.partial(_sc_gather_body, n_ch),
        out_type=(out, out, out, out),
        mesh=mesh,
        scratch_types=[
            idxs, idxs,
            row, row, row, row,
            row, row, row, row,
            pltpu.SemaphoreType.DMA,
            pltpu.SemaphoreType.DMA,
        ],
    )
    return fn(idxl.reshape(NW, n_ch, CH), idxr.reshape(NW, n_ch, CH), h, c)


def _tc_body(Hs, hl_ref, hr_ref, cl_ref, cr_ref, W_ref, b_ref, ho_ref, co_ref):
    hl = hl_ref[...].astype(jnp.bfloat16)
    hr = hr_ref[...].astype(jnp.bfloat16)
    z = (jnp.dot(hl, W_ref[:Hs, :], preferred_element_type=jnp.float32)
         + jnp.dot(hr, W_ref[Hs:, :], preferred_element_type=jnp.float32)
         + b_ref[...])
    f = jax.nn.sigmoid(z[:, :2 * Hs])
    c_sum = f[:, :Hs] * cl_ref[...] + f[:, Hs:] * cr_ref[...]
    i = jax.nn.sigmoid(z[:, 2 * Hs:3 * Hs])
    o = jax.nn.sigmoid(z[:, 3 * Hs:4 * Hs])
    u = jnp.tanh(z[:, 4 * Hs:])
    c_out = i * u + c_sum
    co_ref[...] = c_out
    ho_ref[...] = o * jnp.tanh(c_out)


def _tc_dense(hl, hr, cl, cr, W, b, P, bp):
    Hs = hl.shape[1]
    grid = P // bp
    blk = pl.BlockSpec((bp, Hs), lambda i: (i, 0))
    wblk = pl.BlockSpec(W.shape, lambda i: (0, 0))
    bblk = pl.BlockSpec(b.shape, lambda i: (0, 0))
    out = jax.ShapeDtypeStruct((P, Hs), jnp.float32)
    return pl.pallas_call(
        functools.partial(_tc_body, Hs),
        grid=(grid,),
        in_specs=[blk, blk, blk, blk, wblk, bblk],
        out_specs=[blk, blk],
        out_shape=(out, out),
    )(hl, hr, cl, cr, W, b)


def kernel(h, c, child_idx, U_f, b_f, U_iou, b_iou):
    P = child_idx.shape[0]
    idx = child_idx.astype(jnp.int32)
    step = NW * CH
    P_pad = ((P + step - 1) // step) * step
    n_ch = P_pad // step
    pad = P_pad - P
    idxl = jnp.pad(idx[:, 0], (0, pad))
    idxr = jnp.pad(idx[:, 1], (0, pad))

    hl, hr, cl, cr = _sc_gather(idxl, idxr, h, c, n_ch)

    W = jnp.concatenate([U_f, U_iou], axis=1).astype(jnp.bfloat16)
    b = jnp.concatenate([b_f, b_iou.reshape(-1)]).reshape(1, -1)
    bp = 1000
    while P % bp != 0:
        bp //= 2
    return _tc_dense(hl, hr, cl, cr, W, b, P, bp)
